# Initial kernel scaffold; baseline (speedup 1.0000x reference)
import jax
import jax.numpy as jnp
from jax.experimental import pallas as pl


def kernel(src, tgt, W_src, b_src, W_tgt, b_tgt, W_edge, b_edge, W_node, b_node):
    raise NotImplementedError("write your pallas kernel here")


# speedup vs baseline: 11.6691x; 11.6691x over previous
import functools

import jax
import jax.numpy as jnp
from jax import lax
from jax.experimental import pallas as pl
from jax.experimental.pallas import tpu as pltpu
from jax.experimental.pallas import tpu_sc as plsc

B, N, M, D, K = 4, 2048, 512, 128, 6
P = N + M
G = B * P
DE = 2 * D

NC, NS = 2, 16
NW = NC * NS
GW = G // NW
CN = 16
NCHUNK = GW // CN
CI = CN * K

RK = 256
RF = 512


def _knn_body(x_ref, y_ref, o_ref, *, pcols, rblk):
    x = x_ref[0]
    y = y_ref[0]
    d = (x[:, 0:1] - y[0:1, :]) ** 2
    d = d + (x[:, 1:2] - y[1:2, :]) ** 2
    d = d + (x[:, 2:3] - y[2:3, :]) ** 2
    rid = lax.broadcasted_iota(jnp.int32, (rblk, pcols), 0) + pl.program_id(1) * rblk
    cid = lax.broadcasted_iota(jnp.int32, (rblk, pcols), 1)
    d = jnp.where(cid == rid, 1e10, d)
    for t in range(K):
        m = jnp.min(d, axis=1, keepdims=True)
        am = jnp.min(jnp.where(d == m, cid, pcols), axis=1, keepdims=True)
        o_ref[0, :, t : t + 1] = am
        d = jnp.where(cid == am, 1e10, d)


def _knn(xyz, xyz_t, pcols):
    nblk = pcols // RK
    return pl.pallas_call(
        functools.partial(_knn_body, pcols=pcols, rblk=RK),
        grid=(B, nblk),
        in_specs=[
            pl.BlockSpec((1, RK, 3), lambda b, r: (b, r, 0)),
            pl.BlockSpec((1, 3, pcols), lambda b, r: (b, 0, 0)),
        ],
        out_specs=pl.BlockSpec((1, RK, K), lambda b, r: (b, r, 0)),
        out_shape=jax.ShapeDtypeStruct((B, pcols, K), jnp.int32),
    )(xyz, xyz_t)


def _feat_body(x_ref, ws_ref, bs_ref, wt_ref, bt_ref, wa_ref, wb_ref, be_ref,
               emb_ref, fa_ref, fbb_ref):
    is_tgt = (pl.program_id(0) % (P // RF)) == (N // RF)
    w = jnp.where(is_tgt, wt_ref[...], ws_ref[...])
    bv = jnp.where(is_tgt, bt_ref[...], bs_ref[...])
    emb = jnp.maximum(jnp.dot(x_ref[...], w, preferred_element_type=jnp.float32) + bv, 0.0)
    emb_ref[...] = emb
    fa_ref[...] = jnp.dot(emb, wa_ref[...], preferred_element_type=jnp.float32)
    fbb_ref[...] = (
        jnp.dot(emb, wb_ref[...], preferred_element_type=jnp.float32) + be_ref[...]
    )


def _feat(xyz, w_s, b_s, w_t, b_t, w_a, w_b, b_e):
    full = lambda shape: pl.BlockSpec(shape, lambda i: tuple(0 for _ in shape))
    return pl.pallas_call(
        _feat_body,
        grid=(G // RF,),
        in_specs=[
            pl.BlockSpec((RF, 3), lambda i: (i, 0)),
            full((3, D)), full((1, D)), full((3, D)), full((1, D)),
            full((D, DE)), full((D, DE)), full((1, DE)),
        ],
        out_specs=[
            pl.BlockSpec((RF, D), lambda i: (i, 0)),
            pl.BlockSpec((RF, DE), lambda i: (i, 0)),
            pl.BlockSpec((RF, DE), lambda i: (i, 0)),
        ],
        out_shape=[
            jax.ShapeDtypeStruct((G, D), jnp.float32),
            jax.ShapeDtypeStruct((G, DE), jnp.float32),
            jax.ShapeDtypeStruct((G, DE), jnp.float32),
        ],
    )(xyz, w_s, b_s, w_t, b_t, w_a, w_b, b_e)


def _sc_agg(fa, fbb, nbr):
    mesh = plsc.VectorSubcoreMesh(
        core_axis_name="c", subcore_axis_name="s", num_cores=NC, num_subcores=NS
    )

    @functools.partial(
        pl.kernel,
        mesh=mesh,
        out_type=jax.ShapeDtypeStruct((G, DE), jnp.float32),
        scratch_types=[
            pltpu.VMEM((NCHUNK, CI), jnp.int32),
            pltpu.VMEM((CI, DE), jnp.float32),
            pltpu.VMEM((CN, DE), jnp.float32),
            pltpu.VMEM((CN, DE), jnp.float32),
            pltpu.SemaphoreType.DMA,
        ],
    )
    def body(fa_hbm, fbb_hbm, nbr_hbm, out_hbm, idx_v, rows_v, fbb_v, agg_v, sem):
        wid = lax.axis_index("s") * NC + lax.axis_index("c")
        base = wid * GW
        pltpu.sync_copy(nbr_hbm.at[wid], idx_v)

        def chunk(j, carry):
            row0 = base + j * CN
            pltpu.async_copy(fa_hbm.at[idx_v.at[j]], rows_v, sem).wait()
            pltpu.sync_copy(fbb_hbm.at[pl.ds(row0, CN)], fbb_v)

            def node(n, c2):
                for v in range(DE // 16):
                    sl = pl.ds(v * 16, 16)
                    f = fbb_v[n, sl]
                    acc = jnp.maximum(rows_v[n * K, sl] + f, 0.0)
                    for k in range(1, K):
                        acc = acc + jnp.maximum(rows_v[n * K + k, sl] + f, 0.0)
                    agg_v[n, sl] = acc
                return c2

            lax.fori_loop(0, CN, node, 0)
            pltpu.sync_copy(agg_v, out_hbm.at[pl.ds(row0, CN)])
            return carry

        lax.fori_loop(0, NCHUNK, chunk, 0)

    return body(fa, fbb, nbr)


def _node_body(emb_ref, agg_ref, w1_ref, w2_ref, bn_ref, o_ref):
    h = jnp.dot(emb_ref[...], w1_ref[...], preferred_element_type=jnp.float32)
    h = h + jnp.dot(agg_ref[...], w2_ref[...], preferred_element_type=jnp.float32)
    o_ref[...] = jnp.maximum(h + bn_ref[...], 0.0)


def _node(emb, agg, w1, w2, bn):
    full = lambda shape: pl.BlockSpec(shape, lambda i: tuple(0 for _ in shape))
    return pl.pallas_call(
        _node_body,
        grid=(G // RF,),
        in_specs=[
            pl.BlockSpec((RF, D), lambda i: (i, 0)),
            pl.BlockSpec((RF, DE), lambda i: (i, 0)),
            full((D, D)), full((DE, D)), full((1, D)),
        ],
        out_specs=pl.BlockSpec((RF, D), lambda i: (i, 0)),
        out_shape=jax.ShapeDtypeStruct((G, D), jnp.float32),
    )(emb, agg, w1, w2, bn)


def kernel(src, tgt, W_src, b_src, W_tgt, b_tgt, W_edge, b_edge, W_node, b_node):
    idx_s = _knn(src, jnp.transpose(src, (0, 2, 1)), N)
    idx_t = _knn(tgt, jnp.transpose(tgt, (0, 2, 1)), M)
    nbr = jnp.concatenate([idx_s, idx_t + N], axis=1)
    nbr = nbr + (jnp.arange(B, dtype=jnp.int32) * P)[:, None, None]
    nbr = nbr.reshape(NW, NCHUNK, CI)

    xyz = jnp.concatenate([src, tgt], axis=1).reshape(G, 3)
    emb, fa, fbb = _feat(
        xyz, W_src, b_src.reshape(1, D), W_tgt, b_tgt.reshape(1, D),
        W_edge[:D], W_edge[D:], b_edge.reshape(1, DE),
    )
    agg = _sc_agg(fa, fbb, nbr)
    out = _node(emb, agg, W_node[:D], W_node[D:], b_node.reshape(1, D))
    return out.reshape(B, P, D)


# speedup vs baseline: 13.4473x; 1.1524x over previous
import functools

import jax
import jax.numpy as jnp
from jax import lax
from jax.experimental import pallas as pl
from jax.experimental.pallas import tpu as pltpu
from jax.experimental.pallas import tpu_sc as plsc

B, N, M, D, K = 4, 2048, 512, 128, 6
P = N + M
G = B * P
DE = 2 * D

NC, NS = 2, 16
NW = NC * NS
GW = G // NW
CN = 16
NCHUNK = GW // CN
CI = CN * K

RK = 256
RF = 512


def _knn_body(x_ref, y_ref, o_ref, *, pcols, rblk):
    x = x_ref[0]
    y = y_ref[0]
    d = (x[:, 0:1] - y[0:1, :]) ** 2
    d = d + (x[:, 1:2] - y[1:2, :]) ** 2
    d = d + (x[:, 2:3] - y[2:3, :]) ** 2
    rid = lax.broadcasted_iota(jnp.int32, (rblk, pcols), 0) + pl.program_id(1) * rblk
    cid = lax.broadcasted_iota(jnp.int32, (rblk, pcols), 1)
    d = jnp.where(cid == rid, 1e10, d)
    for t in range(K):
        m = jnp.min(d, axis=1, keepdims=True)
        am = jnp.min(jnp.where(d == m, cid, pcols), axis=1, keepdims=True)
        o_ref[0, :, t : t + 1] = am
        d = jnp.where(cid == am, 1e10, d)


def _knn(xyz, xyz_t, pcols):
    nblk = pcols // RK
    return pl.pallas_call(
        functools.partial(_knn_body, pcols=pcols, rblk=RK),
        grid=(B, nblk),
        in_specs=[
            pl.BlockSpec((1, RK, 3), lambda b, r: (b, r, 0)),
            pl.BlockSpec((1, 3, pcols), lambda b, r: (b, 0, 0)),
        ],
        out_specs=pl.BlockSpec((1, RK, K), lambda b, r: (b, r, 0)),
        out_shape=jax.ShapeDtypeStruct((B, pcols, K), jnp.int32),
    )(xyz, xyz_t)


def _feat_body(x_ref, ws_ref, bs_ref, wt_ref, bt_ref, wa_ref, wb_ref, be_ref,
               emb_ref, fa_ref, fbb_ref):
    is_tgt = (pl.program_id(0) % (P // RF)) == (N // RF)
    w = jnp.where(is_tgt, wt_ref[...], ws_ref[...])
    bv = jnp.where(is_tgt, bt_ref[...], bs_ref[...])
    emb = jnp.maximum(jnp.dot(x_ref[...], w, preferred_element_type=jnp.float32) + bv, 0.0)
    emb_ref[...] = emb
    fa_ref[...] = jnp.dot(emb, wa_ref[...], preferred_element_type=jnp.float32)
    fbb_ref[...] = (
        jnp.dot(emb, wb_ref[...], preferred_element_type=jnp.float32) + be_ref[...]
    )


def _feat(xyz, w_s, b_s, w_t, b_t, w_a, w_b, b_e):
    full = lambda shape: pl.BlockSpec(shape, lambda i: tuple(0 for _ in shape))
    return pl.pallas_call(
        _feat_body,
        grid=(G // RF,),
        in_specs=[
            pl.BlockSpec((RF, 3), lambda i: (i, 0)),
            full((3, D)), full((1, D)), full((3, D)), full((1, D)),
            full((D, DE)), full((D, DE)), full((1, DE)),
        ],
        out_specs=[
            pl.BlockSpec((RF, D), lambda i: (i, 0)),
            pl.BlockSpec((RF, DE), lambda i: (i, 0)),
            pl.BlockSpec((RF, DE), lambda i: (i, 0)),
        ],
        out_shape=[
            jax.ShapeDtypeStruct((G, D), jnp.float32),
            jax.ShapeDtypeStruct((G, DE), jnp.float32),
            jax.ShapeDtypeStruct((G, DE), jnp.float32),
        ],
    )(xyz, w_s, b_s, w_t, b_t, w_a, w_b, b_e)


def _sc_agg(fa, fbb, nbr):
    mesh = plsc.VectorSubcoreMesh(
        core_axis_name="c", subcore_axis_name="s", num_cores=NC, num_subcores=NS
    )

    @functools.partial(
        pl.kernel,
        mesh=mesh,
        out_type=jax.ShapeDtypeStruct((G, DE), jnp.float32),
        scratch_types=[
            pltpu.VMEM((NCHUNK, CI), jnp.int32),
            pltpu.VMEM((2, CI, DE), jnp.float32),
            pltpu.VMEM((2, CN, DE), jnp.float32),
            pltpu.VMEM((CN, DE), jnp.float32),
            pltpu.SemaphoreType.DMA,
            pltpu.SemaphoreType.DMA,
            pltpu.SemaphoreType.DMA,
            pltpu.SemaphoreType.DMA,
        ],
    )
    def body(fa_hbm, fbb_hbm, nbr_hbm, out_hbm, idx_v, rows_v, fbb_v, agg_v,
             g0, g1, f0, f1):
        wid = lax.axis_index("s") * NC + lax.axis_index("c")
        base = wid * GW
        pltpu.sync_copy(nbr_hbm.at[wid], idx_v)
        gsem, fsem = (g0, g1), (f0, f1)

        def start(j, slot):
            pltpu.async_copy(fa_hbm.at[idx_v.at[j]], rows_v.at[slot], gsem[slot])
            pltpu.async_copy(
                fbb_hbm.at[pl.ds(base + j * CN, CN)], fbb_v.at[slot], fsem[slot]
            )

        def wait(j, slot):
            pltpu.make_async_copy(
                fa_hbm.at[idx_v.at[j]], rows_v.at[slot], gsem[slot]
            ).wait()
            pltpu.make_async_copy(
                fbb_hbm.at[pl.ds(base + j * CN, CN)], fbb_v.at[slot], fsem[slot]
            ).wait()

        def compute(j, slot):
            def node(n, c2):
                for v in range(DE // 16):
                    sl = pl.ds(v * 16, 16)
                    f = fbb_v[slot, n, sl]
                    acc = jnp.maximum(rows_v[slot, n * K, sl] + f, 0.0)
                    for k in range(1, K):
                        acc = acc + jnp.maximum(rows_v[slot, n * K + k, sl] + f, 0.0)
                    agg_v[n, sl] = acc
                return c2

            lax.fori_loop(0, CN, node, 0)
            pltpu.sync_copy(agg_v, out_hbm.at[pl.ds(base + j * CN, CN)])

        start(0, 0)

        def pair(g, carry):
            j0 = 2 * g
            start(j0 + 1, 1)
            wait(j0, 0)
            compute(j0, 0)

            @pl.when(g < NCHUNK // 2 - 1)
            def _():
                start(j0 + 2, 0)

            wait(j0 + 1, 1)
            compute(j0 + 1, 1)
            return carry

        lax.fori_loop(0, NCHUNK // 2, pair, 0)

    return body(fa, fbb, nbr)


def _node_body(emb_ref, agg_ref, w1_ref, w2_ref, bn_ref, o_ref):
    h = jnp.dot(emb_ref[...], w1_ref[...], preferred_element_type=jnp.float32)
    h = h + jnp.dot(agg_ref[...], w2_ref[...], preferred_element_type=jnp.float32)
    o_ref[...] = jnp.maximum(h + bn_ref[...], 0.0)


def _node(emb, agg, w1, w2, bn):
    full = lambda shape: pl.BlockSpec(shape, lambda i: tuple(0 for _ in shape))
    return pl.pallas_call(
        _node_body,
        grid=(G // RF,),
        in_specs=[
            pl.BlockSpec((RF, D), lambda i: (i, 0)),
            pl.BlockSpec((RF, DE), lambda i: (i, 0)),
            full((D, D)), full((DE, D)), full((1, D)),
        ],
        out_specs=pl.BlockSpec((RF, D), lambda i: (i, 0)),
        out_shape=jax.ShapeDtypeStruct((G, D), jnp.float32),
    )(emb, agg, w1, w2, bn)


def kernel(src, tgt, W_src, b_src, W_tgt, b_tgt, W_edge, b_edge, W_node, b_node):
    idx_s = _knn(src, jnp.transpose(src, (0, 2, 1)), N)
    idx_t = _knn(tgt, jnp.transpose(tgt, (0, 2, 1)), M)
    nbr = jnp.concatenate([idx_s, idx_t + N], axis=1)
    nbr = nbr + (jnp.arange(B, dtype=jnp.int32) * P)[:, None, None]
    nbr = nbr.reshape(NW, NCHUNK, CI)

    xyz = jnp.concatenate([src, tgt], axis=1).reshape(G, 3)
    emb, fa, fbb = _feat(
        xyz, W_src, b_src.reshape(1, D), W_tgt, b_tgt.reshape(1, D),
        W_edge[:D], W_edge[D:], b_edge.reshape(1, DE),
    )
    agg = _sc_agg(fa, fbb, nbr)
    out = _node(emb, agg, W_node[:D], W_node[D:], b_node.reshape(1, D))
    return out.reshape(B, P, D)


# speedup vs baseline: 15.4593x; 1.1496x over previous
import functools

import jax
import jax.numpy as jnp
from jax import lax
from jax.experimental import pallas as pl
from jax.experimental.pallas import tpu as pltpu
from jax.experimental.pallas import tpu_sc as plsc

B, N, M, D, K = 4, 2048, 512, 128, 6
P = N + M
G = B * P
DE = 2 * D

NC, NS = 2, 16
NW = NC * NS
GW = G // NW
CN = 16
NCHUNK = GW // CN
CI = CN * K

RK = 256
RF = 512


def _knn_body(x_ref, y_ref, o_ref, *, pcols, rblk):
    x = x_ref[0]
    y = y_ref[0]
    d = (x[:, 0:1] - y[0:1, :]) ** 2
    d = d + (x[:, 1:2] - y[1:2, :]) ** 2
    d = d + (x[:, 2:3] - y[2:3, :]) ** 2
    rid = lax.broadcasted_iota(jnp.int32, (rblk, pcols), 0) + pl.program_id(1) * rblk
    cid = lax.broadcasted_iota(jnp.int32, (rblk, pcols), 1)
    d = jnp.where(cid == rid, 1e10, d)
    key = lax.bitwise_or(
        lax.bitwise_and(lax.bitcast_convert_type(d, jnp.int32), -2048), cid
    )
    for t in range(K):
        m = jnp.min(key, axis=1, keepdims=True)
        o_ref[0, :, t : t + 1] = lax.bitwise_and(m, 2047)
        key = jnp.where(key == m, jnp.int32(2**31 - 1), key)


def _knn(xyz, xyz_t, pcols):
    nblk = pcols // RK
    return pl.pallas_call(
        functools.partial(_knn_body, pcols=pcols, rblk=RK),
        grid=(B, nblk),
        in_specs=[
            pl.BlockSpec((1, RK, 3), lambda b, r: (b, r, 0)),
            pl.BlockSpec((1, 3, pcols), lambda b, r: (b, 0, 0)),
        ],
        out_specs=pl.BlockSpec((1, RK, K), lambda b, r: (b, r, 0)),
        out_shape=jax.ShapeDtypeStruct((B, pcols, K), jnp.int32),
    )(xyz, xyz_t)


def _feat_body(x_ref, ws_ref, bs_ref, wt_ref, bt_ref, wa_ref, wb_ref, be_ref,
               emb_ref, fa_ref, fbb_ref):
    is_tgt = (pl.program_id(0) % (P // RF)) == (N // RF)
    w = jnp.where(is_tgt, wt_ref[...], ws_ref[...])
    bv = jnp.where(is_tgt, bt_ref[...], bs_ref[...])
    emb = jnp.maximum(jnp.dot(x_ref[...], w, preferred_element_type=jnp.float32) + bv, 0.0)
    emb_ref[...] = emb
    fa_ref[...] = jnp.dot(emb, wa_ref[...], preferred_element_type=jnp.float32)
    fbb_ref[...] = (
        jnp.dot(emb, wb_ref[...], preferred_element_type=jnp.float32) + be_ref[...]
    )


def _feat(xyz, w_s, b_s, w_t, b_t, w_a, w_b, b_e):
    full = lambda shape: pl.BlockSpec(shape, lambda i: tuple(0 for _ in shape))
    return pl.pallas_call(
        _feat_body,
        grid=(G // RF,),
        in_specs=[
            pl.BlockSpec((RF, 3), lambda i: (i, 0)),
            full((3, D)), full((1, D)), full((3, D)), full((1, D)),
            full((D, DE)), full((D, DE)), full((1, DE)),
        ],
        out_specs=[
            pl.BlockSpec((RF, D), lambda i: (i, 0)),
            pl.BlockSpec((RF, DE), lambda i: (i, 0)),
            pl.BlockSpec((RF, DE), lambda i: (i, 0)),
        ],
        out_shape=[
            jax.ShapeDtypeStruct((G, D), jnp.float32),
            jax.ShapeDtypeStruct((G, DE), jnp.float32),
            jax.ShapeDtypeStruct((G, DE), jnp.float32),
        ],
    )(xyz, w_s, b_s, w_t, b_t, w_a, w_b, b_e)


def _sc_agg(fa, fbb, nbr):
    mesh = plsc.VectorSubcoreMesh(
        core_axis_name="c", subcore_axis_name="s", num_cores=NC, num_subcores=NS
    )

    @functools.partial(
        pl.kernel,
        mesh=mesh,
        out_type=jax.ShapeDtypeStruct((G, DE), jnp.float32),
        scratch_types=[
            pltpu.VMEM((NCHUNK, CI), jnp.int32),
            pltpu.VMEM((2, CI, DE), jnp.float32),
            pltpu.VMEM((2, CN, DE), jnp.float32),
            pltpu.VMEM((CN, DE), jnp.float32),
            pltpu.SemaphoreType.DMA,
            pltpu.SemaphoreType.DMA,
            pltpu.SemaphoreType.DMA,
            pltpu.SemaphoreType.DMA,
        ],
    )
    def body(fa_hbm, fbb_hbm, nbr_hbm, out_hbm, idx_v, rows_v, fbb_v, agg_v,
             g0, g1, f0, f1):
        wid = lax.axis_index("s") * NC + lax.axis_index("c")
        base = wid * GW
        pltpu.sync_copy(nbr_hbm.at[wid], idx_v)
        gsem, fsem = (g0, g1), (f0, f1)

        def start(j, slot):
            pltpu.async_copy(fa_hbm.at[idx_v.at[j]], rows_v.at[slot], gsem[slot])
            pltpu.async_copy(
                fbb_hbm.at[pl.ds(base + j * CN, CN)], fbb_v.at[slot], fsem[slot]
            )

        def wait(j, slot):
            pltpu.make_async_copy(
                fa_hbm.at[idx_v.at[j]], rows_v.at[slot], gsem[slot]
            ).wait()
            pltpu.make_async_copy(
                fbb_hbm.at[pl.ds(base + j * CN, CN)], fbb_v.at[slot], fsem[slot]
            ).wait()

        def compute(j, slot):
            def node(n, c2):
                for v in range(DE // 16):
                    sl = pl.ds(v * 16, 16)
                    f = fbb_v[slot, n, sl]
                    acc = jnp.maximum(rows_v[slot, n * K, sl] + f, 0.0)
                    for k in range(1, K):
                        acc = acc + jnp.maximum(rows_v[slot, n * K + k, sl] + f, 0.0)
                    agg_v[n, sl] = acc
                return c2

            lax.fori_loop(0, CN, node, 0)
            pltpu.sync_copy(agg_v, out_hbm.at[pl.ds(base + j * CN, CN)])

        start(0, 0)

        def pair(g, carry):
            j0 = 2 * g
            start(j0 + 1, 1)
            wait(j0, 0)
            compute(j0, 0)

            @pl.when(g < NCHUNK // 2 - 1)
            def _():
                start(j0 + 2, 0)

            wait(j0 + 1, 1)
            compute(j0 + 1, 1)
            return carry

        lax.fori_loop(0, NCHUNK // 2, pair, 0)

    return body(fa, fbb, nbr)


def _node_body(emb_ref, agg_ref, w1_ref, w2_ref, bn_ref, o_ref):
    h = jnp.dot(emb_ref[...], w1_ref[...], preferred_element_type=jnp.float32)
    h = h + jnp.dot(agg_ref[...], w2_ref[...], preferred_element_type=jnp.float32)
    o_ref[...] = jnp.maximum(h + bn_ref[...], 0.0)


def _node(emb, agg, w1, w2, bn):
    full = lambda shape: pl.BlockSpec(shape, lambda i: tuple(0 for _ in shape))
    return pl.pallas_call(
        _node_body,
        grid=(G // RF,),
        in_specs=[
            pl.BlockSpec((RF, D), lambda i: (i, 0)),
            pl.BlockSpec((RF, DE), lambda i: (i, 0)),
            full((D, D)), full((DE, D)), full((1, D)),
        ],
        out_specs=pl.BlockSpec((RF, D), lambda i: (i, 0)),
        out_shape=jax.ShapeDtypeStruct((G, D), jnp.float32),
    )(emb, agg, w1, w2, bn)


def kernel(src, tgt, W_src, b_src, W_tgt, b_tgt, W_edge, b_edge, W_node, b_node):
    idx_s = _knn(src, jnp.transpose(src, (0, 2, 1)), N)
    idx_t = _knn(tgt, jnp.transpose(tgt, (0, 2, 1)), M)
    nbr = jnp.concatenate([idx_s, idx_t + N], axis=1)
    nbr = nbr + (jnp.arange(B, dtype=jnp.int32) * P)[:, None, None]
    nbr = nbr.reshape(NW, NCHUNK, CI)

    xyz = jnp.concatenate([src, tgt], axis=1).reshape(G, 3)
    emb, fa, fbb = _feat(
        xyz, W_src, b_src.reshape(1, D), W_tgt, b_tgt.reshape(1, D),
        W_edge[:D], W_edge[D:], b_edge.reshape(1, DE),
    )
    agg = _sc_agg(fa, fbb, nbr)
    out = _node(emb, agg, W_node[:D], W_node[D:], b_node.reshape(1, D))
    return out.reshape(B, P, D)


# speedup vs baseline: 18.4131x; 1.1911x over previous
import functools

import jax
import jax.numpy as jnp
from jax import lax
from jax.experimental import pallas as pl
from jax.experimental.pallas import tpu as pltpu
from jax.experimental.pallas import tpu_sc as plsc

B, N, M, D, K = 4, 2048, 512, 128, 6
P = N + M
G = B * P
DE = 2 * D

NC, NS = 2, 16
NW = NC * NS
GW = G // NW
CN = 16
NCHUNK = GW // CN
CI = CN * K

RK = 256
RF = 512


def _knn_body(x_ref, y_ref, o_ref, *, pcols, rblk):
    x = x_ref[0]
    y = y_ref[0]
    xy2 = jnp.dot(x * -2.0, y, preferred_element_type=jnp.float32)
    ny = jnp.sum(y * y, axis=0, keepdims=True)
    nx = jnp.sum(x * x, axis=1, keepdims=True)
    d = jnp.maximum((xy2 + ny) + nx, 0.0)
    rid = lax.broadcasted_iota(jnp.int32, (rblk, pcols), 0) + pl.program_id(1) * rblk
    cid = lax.broadcasted_iota(jnp.int32, (rblk, pcols), 1)
    d = jnp.where(cid == rid, 1e10, d)
    key = lax.bitcast_convert_type(
        lax.bitwise_or(
            lax.bitwise_and(lax.bitcast_convert_type(d, jnp.int32), -2048), cid
        ),
        jnp.float32,
    )
    for t in range(K):
        m = jnp.min(key, axis=1, keepdims=True)
        o_ref[0, :, t : t + 1] = lax.bitwise_and(
            lax.bitcast_convert_type(m, jnp.int32), 2047
        )
        key = jnp.where(key == m, jnp.float32(jnp.inf), key)


def _knn(xyz, xyz_t, pcols):
    nblk = pcols // RK
    return pl.pallas_call(
        functools.partial(_knn_body, pcols=pcols, rblk=RK),
        grid=(B, nblk),
        in_specs=[
            pl.BlockSpec((1, RK, 3), lambda b, r: (b, r, 0)),
            pl.BlockSpec((1, 3, pcols), lambda b, r: (b, 0, 0)),
        ],
        out_specs=pl.BlockSpec((1, RK, K), lambda b, r: (b, r, 0)),
        out_shape=jax.ShapeDtypeStruct((B, pcols, K), jnp.int32),
    )(xyz, xyz_t)


def _feat_body(x_ref, ws_ref, bs_ref, wt_ref, bt_ref, wa_ref, wb_ref, be_ref,
               emb_ref, fa_ref, fbb_ref):
    is_tgt = (pl.program_id(0) % (P // RF)) == (N // RF)
    w = jnp.where(is_tgt, wt_ref[...], ws_ref[...])
    bv = jnp.where(is_tgt, bt_ref[...], bs_ref[...])
    emb = jnp.maximum(jnp.dot(x_ref[...], w, preferred_element_type=jnp.float32) + bv, 0.0)
    emb_ref[...] = emb
    fa_ref[...] = jnp.dot(emb, wa_ref[...], preferred_element_type=jnp.float32)
    fbb_ref[...] = (
        jnp.dot(emb, wb_ref[...], preferred_element_type=jnp.float32) + be_ref[...]
    )


def _feat(xyz, w_s, b_s, w_t, b_t, w_a, w_b, b_e):
    full = lambda shape: pl.BlockSpec(shape, lambda i: tuple(0 for _ in shape))
    return pl.pallas_call(
        _feat_body,
        grid=(G // RF,),
        in_specs=[
            pl.BlockSpec((RF, 3), lambda i: (i, 0)),
            full((3, D)), full((1, D)), full((3, D)), full((1, D)),
            full((D, DE)), full((D, DE)), full((1, DE)),
        ],
        out_specs=[
            pl.BlockSpec((RF, D), lambda i: (i, 0)),
            pl.BlockSpec((RF, DE), lambda i: (i, 0)),
            pl.BlockSpec((RF, DE), lambda i: (i, 0)),
        ],
        out_shape=[
            jax.ShapeDtypeStruct((G, D), jnp.float32),
            jax.ShapeDtypeStruct((G, DE), jnp.float32),
            jax.ShapeDtypeStruct((G, DE), jnp.float32),
        ],
    )(xyz, w_s, b_s, w_t, b_t, w_a, w_b, b_e)


def _sc_agg(fa, fbb, nbr):
    mesh = plsc.VectorSubcoreMesh(
        core_axis_name="c", subcore_axis_name="s", num_cores=NC, num_subcores=NS
    )

    @functools.partial(
        pl.kernel,
        mesh=mesh,
        out_type=jax.ShapeDtypeStruct((G, DE), jnp.float32),
        scratch_types=[
            pltpu.VMEM((NCHUNK, CI), jnp.int32),
            pltpu.VMEM((2, CI, DE), jnp.float32),
            pltpu.VMEM((2, CN, DE), jnp.float32),
            pltpu.VMEM((CN, DE), jnp.float32),
            pltpu.SemaphoreType.DMA,
            pltpu.SemaphoreType.DMA,
            pltpu.SemaphoreType.DMA,
            pltpu.SemaphoreType.DMA,
        ],
    )
    def body(fa_hbm, fbb_hbm, nbr_hbm, out_hbm, idx_v, rows_v, fbb_v, agg_v,
             g0, g1, f0, f1):
        wid = lax.axis_index("s") * NC + lax.axis_index("c")
        base = wid * GW
        pltpu.sync_copy(nbr_hbm.at[wid], idx_v)
        gsem, fsem = (g0, g1), (f0, f1)

        def start(j, slot):
            pltpu.async_copy(fa_hbm.at[idx_v.at[j]], rows_v.at[slot], gsem[slot])
            pltpu.async_copy(
                fbb_hbm.at[pl.ds(base + j * CN, CN)], fbb_v.at[slot], fsem[slot]
            )

        def wait(j, slot):
            pltpu.make_async_copy(
                fa_hbm.at[idx_v.at[j]], rows_v.at[slot], gsem[slot]
            ).wait()
            pltpu.make_async_copy(
                fbb_hbm.at[pl.ds(base + j * CN, CN)], fbb_v.at[slot], fsem[slot]
            ).wait()

        def compute(j, slot):
            def node(n, c2):
                for v in range(DE // 16):
                    sl = pl.ds(v * 16, 16)
                    f = fbb_v[slot, n, sl]
                    acc = jnp.maximum(rows_v[slot, n * K, sl] + f, 0.0)
                    for k in range(1, K):
                        acc = acc + jnp.maximum(rows_v[slot, n * K + k, sl] + f, 0.0)
                    agg_v[n, sl] = acc
                return c2

            lax.fori_loop(0, CN, node, 0)
            pltpu.sync_copy(agg_v, out_hbm.at[pl.ds(base + j * CN, CN)])

        start(0, 0)

        def pair(g, carry):
            j0 = 2 * g
            start(j0 + 1, 1)
            wait(j0, 0)
            compute(j0, 0)

            @pl.when(g < NCHUNK // 2 - 1)
            def _():
                start(j0 + 2, 0)

            wait(j0 + 1, 1)
            compute(j0 + 1, 1)
            return carry

        lax.fori_loop(0, NCHUNK // 2, pair, 0)

    return body(fa, fbb, nbr)


def _node_body(emb_ref, agg_ref, w1_ref, w2_ref, bn_ref, o_ref):
    h = jnp.dot(emb_ref[...], w1_ref[...], preferred_element_type=jnp.float32)
    h = h + jnp.dot(agg_ref[...], w2_ref[...], preferred_element_type=jnp.float32)
    o_ref[...] = jnp.maximum(h + bn_ref[...], 0.0)


def _node(emb, agg, w1, w2, bn):
    full = lambda shape: pl.BlockSpec(shape, lambda i: tuple(0 for _ in shape))
    return pl.pallas_call(
        _node_body,
        grid=(G // RF,),
        in_specs=[
            pl.BlockSpec((RF, D), lambda i: (i, 0)),
            pl.BlockSpec((RF, DE), lambda i: (i, 0)),
            full((D, D)), full((DE, D)), full((1, D)),
        ],
        out_specs=pl.BlockSpec((RF, D), lambda i: (i, 0)),
        out_shape=jax.ShapeDtypeStruct((G, D), jnp.float32),
    )(emb, agg, w1, w2, bn)


def kernel(src, tgt, W_src, b_src, W_tgt, b_tgt, W_edge, b_edge, W_node, b_node):
    idx_s = _knn(src, jnp.transpose(src, (0, 2, 1)), N)
    idx_t = _knn(tgt, jnp.transpose(tgt, (0, 2, 1)), M)
    nbr = jnp.concatenate([idx_s, idx_t + N], axis=1)
    nbr = nbr + (jnp.arange(B, dtype=jnp.int32) * P)[:, None, None]
    nbr = nbr.reshape(NW, NCHUNK, CI)

    xyz = jnp.concatenate([src, tgt], axis=1).reshape(G, 3)
    emb, fa, fbb = _feat(
        xyz, W_src, b_src.reshape(1, D), W_tgt, b_tgt.reshape(1, D),
        W_edge[:D], W_edge[D:], b_edge.reshape(1, DE),
    )
    agg = _sc_agg(fa, fbb, nbr)
    out = _node(emb, agg, W_node[:D], W_node[D:], b_node.reshape(1, D))
    return out.reshape(B, P, D)
